# baseline (speedup 1.0000x reference)
import functools

import jax
import jax.numpy as jnp
from jax import lax
from jax.experimental import pallas as pl
from jax.experimental.pallas import tpu as pltpu
from jax.experimental.pallas import tpu_sc as plsc

N = 10000
NROWS = 10240
D = 128
E = 320000
NC = 2
NS = 16
NW = NC * NS
EPAD = 327680
NG = 128
DEGW = 128
BLK = 512
GRID = NROWS // BLK
NBUF = 4
IBLK = 8
STEPS_FAST = 136
STEPS_SLOW = 24
DSTEPS = 80

_MESH = plsc.VectorSubcoreMesh(
    core_axis_name="c", subcore_axis_name="s", num_cores=NC, num_subcores=NS
)
_SLAB = NROWS // NS


def _deg_body(ones_hbm, dst_hbm, out_hbm, dstv, onesv, zbuf, acc, sem):
    c = lax.axis_index("c")
    s = lax.axis_index("s")
    wid = c * NS + s
    zero16 = jnp.zeros((16,), jnp.float32)
    for i in range(16):
        for gdx in range(DEGW // 16):
            zbuf[i, pl.ds(gdx * 16, 16)] = zero16
    pltpu.sync_copy(ones_hbm, onesv)
    for k in range(_SLAB // 16):
        pltpu.sync_copy(zbuf, acc.at[pl.ds(s * _SLAB + k * 16, 16)])
    plsc.subcore_barrier()
    pltpu.sync_copy(dst_hbm.at[pl.ds(wid * DSTEPS, DSTEPS)], dstv)

    def scat(j, p):
        return pltpu.make_async_copy(onesv, acc.at[dstv.at[j]], sem.at[p])

    def step(j, carry):
        p = lax.rem(j, NBUF)
        pltpu.async_copy(onesv, acc.at[dstv.at[j]], sem.at[p], add=True)

        @pl.when(j >= NBUF - 1)
        def _():
            scat(j - (NBUF - 1), lax.rem(j + 1, NBUF)).wait()

        return carry

    lax.fori_loop(0, DSTEPS, step, 0)
    for t in range(NBUF - 1):
        j = DSTEPS - (NBUF - 1) + t
        scat(j, j % NBUF).wait()
    plsc.subcore_barrier()
    pltpu.sync_copy(acc.at[pl.ds(s * _SLAB, _SLAB)],
                    out_hbm.at[c].at[pl.ds(s * _SLAB, _SLAB)])


def _scat_body(g_hbm, src_hbm, dst_hbm, out_hbm, srcv, dstv, rows, zbuf, acc,
               gsem, ssem, isem):
    c = lax.axis_index("c")
    s = lax.axis_index("s")
    zero16 = jnp.zeros((16,), jnp.float32)
    for i in range(16):
        for gdx in range(D // 16):
            zbuf[i, pl.ds(gdx * 16, 16)] = zero16
    for k in range(_SLAB // 16):
        pltpu.sync_copy(zbuf, acc.at[pl.ds(s * _SLAB + k * 16, 16)])
    plsc.subcore_barrier()
    steps = jnp.where(c == 0, STEPS_FAST, STEPS_SLOW)
    base = jnp.where(c == 0, s * STEPS_FAST,
                     NS * STEPS_FAST + s * STEPS_SLOW)

    def idx_fetch(b):
        sl = lax.rem(b, 3)
        pltpu.async_copy(src_hbm.at[pl.ds(base + b * IBLK, IBLK)],
                         srcv.at[sl], isem.at[sl, 0])
        pltpu.async_copy(dst_hbm.at[pl.ds(base + b * IBLK, IBLK)],
                         dstv.at[sl], isem.at[sl, 1])

    def idx_wait(b):
        sl = lax.rem(b, 3)
        pltpu.make_async_copy(src_hbm.at[pl.ds(base + b * IBLK, IBLK)],
                              srcv.at[sl], isem.at[sl, 0]).wait()
        pltpu.make_async_copy(dst_hbm.at[pl.ds(base + b * IBLK, IBLK)],
                              dstv.at[sl], isem.at[sl, 1]).wait()

    def srow(j):
        return srcv.at[lax.rem(j // IBLK, 3)].at[lax.rem(j, IBLK)]

    def drow(j):
        return dstv.at[lax.rem(j // IBLK, 3)].at[lax.rem(j, IBLK)]

    def gath(j, p):
        return pltpu.make_async_copy(g_hbm.at[srow(j)], rows.at[p],
                                     gsem.at[p])

    def scat(j, p):
        return pltpu.make_async_copy(rows.at[p], acc.at[drow(j)], ssem.at[p])

    for b in range(3):
        idx_fetch(b)
    idx_wait(0)
    pltpu.async_copy(g_hbm.at[srow(0)], rows.at[0], gsem.at[0])

    def step(j, carry):
        p = lax.rem(j, 2)
        gath(j, p).wait()
        pltpu.async_copy(rows.at[p], acc.at[drow(j)], ssem.at[p], add=True)

        @pl.when(j + 1 < steps)
        def _():
            pn = 1 - p

            @pl.when(j >= 1)
            def _():
                scat(j - 1, pn).wait()

            @pl.when((lax.rem(j, IBLK) == 0)
                     & (j // IBLK + 2 < steps // IBLK) & (j > 0))
            def _():
                idx_fetch(j // IBLK + 2)

            @pl.when(lax.rem(j + 1, IBLK) == 0)
            def _():
                idx_wait((j + 1) // IBLK)

            pltpu.async_copy(g_hbm.at[srow(j + 1)], rows.at[pn], gsem.at[pn])

        return carry

    lax.fori_loop(0, steps, step, 0)
    scat(steps - 2, 0).wait()
    scat(steps - 1, 1).wait()
    plsc.subcore_barrier()
    pltpu.sync_copy(acc.at[pl.ds(s * _SLAB, _SLAB)],
                    out_hbm.at[c].at[pl.ds(s * _SLAB, _SLAB)])


_deg_kernel = functools.partial(
    pl.kernel, _deg_body,
    out_type=jax.ShapeDtypeStruct((NC, NROWS, DEGW), jnp.float32),
    mesh=_MESH,
    scratch_types=[
        pltpu.VMEM((DSTEPS, 128), jnp.int32),
        pltpu.VMEM((128, DEGW), jnp.float32),
        pltpu.VMEM((16, DEGW), jnp.float32),
        pltpu.VMEM_SHARED((NROWS, DEGW), jnp.float32),
        pltpu.SemaphoreType.DMA((NBUF,)),
    ],
)()

_scat_kernel = functools.partial(
    pl.kernel, _scat_body,
    out_type=jax.ShapeDtypeStruct((NC, NROWS, D), jnp.float32),
    mesh=_MESH,
    scratch_types=[
        pltpu.VMEM((3, IBLK, 128), jnp.int32),
        pltpu.VMEM((3, IBLK, 128), jnp.int32),
        pltpu.VMEM((2, 128, D), jnp.float32),
        pltpu.VMEM((16, D), jnp.float32),
        pltpu.VMEM_SHARED((NROWS, D), jnp.float32),
        pltpu.SemaphoreType.DMA((2,)),
        pltpu.SemaphoreType.DMA((2,)),
        pltpu.SemaphoreType.DMA((3, 2)),
    ],
)()


_PREC = lax.Precision.HIGHEST


def _dis_blk(dp_ref):
    deg = dp_ref[0, :, 0:1] + dp_ref[1, :, 0:1] + 1.0
    return lax.rsqrt(deg)


def _tc1_body(dp_ref, x_ref, w_ref, g_ref):
    dis = _dis_blk(dp_ref)
    g_ref[...] = jnp.dot(x_ref[...], w_ref[...], precision=_PREC,
                         preferred_element_type=jnp.float32) * dis


def _tc_mid_body(dp_ref, s_ref, g_ref, b_ref, w_ref, gn_ref):
    dis = _dis_blk(dp_ref)
    h = (s_ref[0] + s_ref[1] + g_ref[...]) * dis + b_ref[...]
    z = jnp.maximum(h, 0.0)
    gn_ref[...] = jnp.dot(z, w_ref[...], precision=_PREC,
                          preferred_element_type=jnp.float32) * dis


def _tc_fin_body(dp_ref, s_ref, g_ref, b_ref, bat_ref, wl_ref, bl_ref,
                 emb_ref, log_ref, cnt_ref):
    i = pl.program_id(0)
    dis = _dis_blk(dp_ref)
    h = (s_ref[0] + s_ref[1] + g_ref[...]) * dis + b_ref[...]
    gid = lax.broadcasted_iota(jnp.int32, (BLK, NG), 1)
    p = (gid == bat_ref[...]).astype(jnp.float32)
    part = lax.dot_general(p, h, (((0,), (0,)), ((), ())), precision=_PREC,
                           preferred_element_type=jnp.float32)
    ones = jnp.ones((BLK, 8), jnp.float32)
    cpart = lax.dot_general(p, ones, (((0,), (0,)), ((), ())),
                            preferred_element_type=jnp.float32)

    @pl.when(i == 0)
    def _():
        emb_ref[...] = jnp.zeros_like(emb_ref)
        cnt_ref[...] = jnp.zeros_like(cnt_ref)

    emb_ref[...] += part
    cnt_ref[...] += cpart

    @pl.when(i == GRID - 1)
    def _():
        cnt = jnp.maximum(cnt_ref[:, 0:1], 1.0)
        mean = emb_ref[...] / cnt
        emb_ref[...] = mean
        log_ref[...] = jnp.dot(mean, wl_ref[...], precision=_PREC,
                               preferred_element_type=jnp.float32) + bl_ref[...]


def _dp_spec():
    return pl.BlockSpec((NC, BLK, DEGW), lambda i: (0, i, 0))


def _row_spec():
    return pl.BlockSpec((BLK, D), lambda i: (i, 0))


def _s_spec():
    return pl.BlockSpec((NC, BLK, D), lambda i: (0, i, 0))


def _full_spec(r, c):
    return pl.BlockSpec((r, c), lambda i: (0, 0))


_tc1 = pl.pallas_call(
    _tc1_body,
    grid=(GRID,),
    in_specs=[_dp_spec(), _row_spec(), _full_spec(D, D)],
    out_specs=_row_spec(),
    out_shape=jax.ShapeDtypeStruct((NROWS, D), jnp.float32),
    compiler_params=pltpu.CompilerParams(dimension_semantics=("arbitrary",)),
)

_tc_mid = pl.pallas_call(
    _tc_mid_body,
    grid=(GRID,),
    in_specs=[_dp_spec(), _s_spec(), _row_spec(), _full_spec(1, D),
              _full_spec(D, D)],
    out_specs=_row_spec(),
    out_shape=jax.ShapeDtypeStruct((NROWS, D), jnp.float32),
    compiler_params=pltpu.CompilerParams(dimension_semantics=("arbitrary",)),
)

_tc_fin = pl.pallas_call(
    _tc_fin_body,
    grid=(GRID,),
    in_specs=[_dp_spec(), _s_spec(), _row_spec(), _full_spec(1, D),
              pl.BlockSpec((BLK, 1), lambda i: (i, 0)),
              _full_spec(D, D), _full_spec(1, D)],
    out_specs=[_full_spec(NG, D), _full_spec(NG, D)],
    out_shape=[jax.ShapeDtypeStruct((NG, D), jnp.float32),
               jax.ShapeDtypeStruct((NG, D), jnp.float32)],
    scratch_shapes=[pltpu.VMEM((NG, 8), jnp.float32)],
    compiler_params=pltpu.CompilerParams(dimension_semantics=("arbitrary",)),
)


def kernel(x, edge_index, batch, W1, b1, W2, b2, W3, b3, Wlin, blin):
    src = edge_index[0].astype(jnp.int32)
    dst = edge_index[1].astype(jnp.int32)
    srcp = jnp.concatenate([src, jnp.zeros((EPAD - E,), jnp.int32)])
    dstp = jnp.concatenate([dst, jnp.full((EPAD - E,), N, jnp.int32)])
    src128 = srcp.reshape(EPAD // 128, 128)
    dst128 = dstp.reshape(EPAD // 128, 128)
    xp = jnp.pad(x, ((0, NROWS - N), (0, 0)))
    batp = jnp.concatenate(
        [batch.astype(jnp.int32),
         jnp.full((NROWS - N,), -1, jnp.int32)]).reshape(NROWS, 1)
    b1r = b1.reshape(1, D)
    b2r = b2.reshape(1, D)
    b3r = b3.reshape(1, D)
    wl = jnp.pad(Wlin, ((0, 0), (0, D - Wlin.shape[1])))
    blr = jnp.pad(blin, (0, D - blin.shape[0])).reshape(1, D)

    dp = _deg_kernel(jnp.ones((128, DEGW), jnp.float32), dst128)
    g1 = _tc1(dp, xp, W1)
    s1 = _scat_kernel(g1, src128, dst128)
    g2 = _tc_mid(dp, s1, g1, b1r, W2)
    s2 = _scat_kernel(g2, src128, dst128)
    g3 = _tc_mid(dp, s2, g2, b2r, W3)
    s3 = _scat_kernel(g3, src128, dst128)
    emb, logp = _tc_fin(dp, s3, g3, b3r, batp, wl, blr)
    return (logp[:, :Wlin.shape[1]], emb)


# speedup vs baseline: 1.0279x; 1.0279x over previous
import functools

import jax
import jax.numpy as jnp
from jax import lax
from jax.experimental import pallas as pl
from jax.experimental.pallas import tpu as pltpu
from jax.experimental.pallas import tpu_sc as plsc

N = 10000
NROWS = 10240
D = 128
E = 320000
NC = 2
NS = 16
NW = NC * NS
EPAD = 327680
NG = 128
DEGW = 128
BLK = 512
GRID = NROWS // BLK
NBUF = 4
IBLK = 8
STEPS_FAST = 144
STEPS_SLOW = 16
DSTEPS = 80

_MESH = plsc.VectorSubcoreMesh(
    core_axis_name="c", subcore_axis_name="s", num_cores=NC, num_subcores=NS
)
_SLAB = NROWS // NS


def _deg_body(ones_hbm, dst_hbm, out_hbm, dstv, onesv, zbuf, acc, sem):
    c = lax.axis_index("c")
    s = lax.axis_index("s")
    wid = c * NS + s
    zero16 = jnp.zeros((16,), jnp.float32)
    for i in range(16):
        for gdx in range(DEGW // 16):
            zbuf[i, pl.ds(gdx * 16, 16)] = zero16
    pltpu.sync_copy(ones_hbm, onesv)
    for k in range(_SLAB // 16):
        pltpu.sync_copy(zbuf, acc.at[pl.ds(s * _SLAB + k * 16, 16)])
    plsc.subcore_barrier()
    pltpu.sync_copy(dst_hbm.at[pl.ds(wid * DSTEPS, DSTEPS)], dstv)

    def scat(j, p):
        return pltpu.make_async_copy(onesv, acc.at[dstv.at[j]], sem.at[p])

    def step(j, carry):
        p = lax.rem(j, NBUF)
        pltpu.async_copy(onesv, acc.at[dstv.at[j]], sem.at[p], add=True)

        @pl.when(j >= NBUF - 1)
        def _():
            scat(j - (NBUF - 1), lax.rem(j + 1, NBUF)).wait()

        return carry

    lax.fori_loop(0, DSTEPS, step, 0)
    for t in range(NBUF - 1):
        j = DSTEPS - (NBUF - 1) + t
        scat(j, j % NBUF).wait()
    plsc.subcore_barrier()
    pltpu.sync_copy(acc.at[pl.ds(s * _SLAB, _SLAB)],
                    out_hbm.at[c].at[pl.ds(s * _SLAB, _SLAB)])


def _scat_body(g_hbm, src_hbm, dst_hbm, out_hbm, srcv, dstv, rows, zbuf, acc,
               gsem, ssem, isem):
    c = lax.axis_index("c")
    s = lax.axis_index("s")
    zero16 = jnp.zeros((16,), jnp.float32)
    for i in range(16):
        for gdx in range(D // 16):
            zbuf[i, pl.ds(gdx * 16, 16)] = zero16
    for k in range(_SLAB // 16):
        pltpu.sync_copy(zbuf, acc.at[pl.ds(s * _SLAB + k * 16, 16)])
    plsc.subcore_barrier()
    steps = jnp.where(c == 0, STEPS_FAST, STEPS_SLOW)
    base = jnp.where(c == 0, s * STEPS_FAST,
                     NS * STEPS_FAST + s * STEPS_SLOW)

    def idx_fetch(b):
        sl = lax.rem(b, 3)
        pltpu.async_copy(src_hbm.at[pl.ds(base + b * IBLK, IBLK)],
                         srcv.at[sl], isem.at[sl, 0])
        pltpu.async_copy(dst_hbm.at[pl.ds(base + b * IBLK, IBLK)],
                         dstv.at[sl], isem.at[sl, 1])

    def idx_wait(b):
        sl = lax.rem(b, 3)
        pltpu.make_async_copy(src_hbm.at[pl.ds(base + b * IBLK, IBLK)],
                              srcv.at[sl], isem.at[sl, 0]).wait()
        pltpu.make_async_copy(dst_hbm.at[pl.ds(base + b * IBLK, IBLK)],
                              dstv.at[sl], isem.at[sl, 1]).wait()

    def srow(j):
        return srcv.at[lax.rem(j // IBLK, 3)].at[lax.rem(j, IBLK)]

    def drow(j):
        return dstv.at[lax.rem(j // IBLK, 3)].at[lax.rem(j, IBLK)]

    def gath(j, p):
        return pltpu.make_async_copy(g_hbm.at[srow(j)], rows.at[p],
                                     gsem.at[p])

    def scat(j, p):
        return pltpu.make_async_copy(rows.at[p], acc.at[drow(j)], ssem.at[p])

    for b in range(3):
        idx_fetch(b)
    idx_wait(0)
    pltpu.async_copy(g_hbm.at[srow(0)], rows.at[0], gsem.at[0])

    def step(j, carry):
        p = lax.rem(j, 2)
        gath(j, p).wait()
        pltpu.async_copy(rows.at[p], acc.at[drow(j)], ssem.at[p], add=True)

        @pl.when(j + 1 < steps)
        def _():
            pn = 1 - p

            @pl.when(j >= 1)
            def _():
                scat(j - 1, pn).wait()

            @pl.when((lax.rem(j, IBLK) == 0)
                     & (j // IBLK + 2 < steps // IBLK) & (j > 0))
            def _():
                idx_fetch(j // IBLK + 2)

            @pl.when(lax.rem(j + 1, IBLK) == 0)
            def _():
                idx_wait((j + 1) // IBLK)

            pltpu.async_copy(g_hbm.at[srow(j + 1)], rows.at[pn], gsem.at[pn])

        return carry

    lax.fori_loop(0, steps, step, 0)
    scat(steps - 2, 0).wait()
    scat(steps - 1, 1).wait()
    plsc.subcore_barrier()
    pltpu.sync_copy(acc.at[pl.ds(s * _SLAB, _SLAB)],
                    out_hbm.at[c].at[pl.ds(s * _SLAB, _SLAB)])


_deg_kernel = functools.partial(
    pl.kernel, _deg_body,
    out_type=jax.ShapeDtypeStruct((NC, NROWS, DEGW), jnp.float32),
    mesh=_MESH,
    scratch_types=[
        pltpu.VMEM((DSTEPS, 128), jnp.int32),
        pltpu.VMEM((128, DEGW), jnp.float32),
        pltpu.VMEM((16, DEGW), jnp.float32),
        pltpu.VMEM_SHARED((NROWS, DEGW), jnp.float32),
        pltpu.SemaphoreType.DMA((NBUF,)),
    ],
)()

_scat_kernel = functools.partial(
    pl.kernel, _scat_body,
    out_type=jax.ShapeDtypeStruct((NC, NROWS, D), jnp.float32),
    mesh=_MESH,
    scratch_types=[
        pltpu.VMEM((3, IBLK, 128), jnp.int32),
        pltpu.VMEM((3, IBLK, 128), jnp.int32),
        pltpu.VMEM((2, 128, D), jnp.float32),
        pltpu.VMEM((16, D), jnp.float32),
        pltpu.VMEM_SHARED((NROWS, D), jnp.float32),
        pltpu.SemaphoreType.DMA((2,)),
        pltpu.SemaphoreType.DMA((2,)),
        pltpu.SemaphoreType.DMA((3, 2)),
    ],
)()


_PREC = lax.Precision.HIGHEST


def _dis_blk(dp_ref):
    deg = dp_ref[0, :, 0:1] + dp_ref[1, :, 0:1] + 1.0
    return lax.rsqrt(deg)


def _tc1_body(dp_ref, x_ref, w_ref, g_ref):
    dis = _dis_blk(dp_ref)
    g_ref[...] = jnp.dot(x_ref[...], w_ref[...], precision=_PREC,
                         preferred_element_type=jnp.float32) * dis


def _tc_mid_body(dp_ref, s_ref, g_ref, b_ref, w_ref, gn_ref):
    dis = _dis_blk(dp_ref)
    h = (s_ref[0] + s_ref[1] + g_ref[...]) * dis + b_ref[...]
    z = jnp.maximum(h, 0.0)
    gn_ref[...] = jnp.dot(z, w_ref[...], precision=_PREC,
                          preferred_element_type=jnp.float32) * dis


def _tc_fin_body(dp_ref, s_ref, g_ref, b_ref, bat_ref, wl_ref, bl_ref,
                 emb_ref, log_ref, cnt_ref):
    i = pl.program_id(0)
    dis = _dis_blk(dp_ref)
    h = (s_ref[0] + s_ref[1] + g_ref[...]) * dis + b_ref[...]
    gid = lax.broadcasted_iota(jnp.int32, (BLK, NG), 1)
    p = (gid == bat_ref[...]).astype(jnp.float32)
    part = lax.dot_general(p, h, (((0,), (0,)), ((), ())), precision=_PREC,
                           preferred_element_type=jnp.float32)
    ones = jnp.ones((BLK, 8), jnp.float32)
    cpart = lax.dot_general(p, ones, (((0,), (0,)), ((), ())),
                            preferred_element_type=jnp.float32)

    @pl.when(i == 0)
    def _():
        emb_ref[...] = jnp.zeros_like(emb_ref)
        cnt_ref[...] = jnp.zeros_like(cnt_ref)

    emb_ref[...] += part
    cnt_ref[...] += cpart

    @pl.when(i == GRID - 1)
    def _():
        cnt = jnp.maximum(cnt_ref[:, 0:1], 1.0)
        mean = emb_ref[...] / cnt
        emb_ref[...] = mean
        log_ref[...] = jnp.dot(mean, wl_ref[...], precision=_PREC,
                               preferred_element_type=jnp.float32) + bl_ref[...]


def _dp_spec():
    return pl.BlockSpec((NC, BLK, DEGW), lambda i: (0, i, 0))


def _row_spec():
    return pl.BlockSpec((BLK, D), lambda i: (i, 0))


def _s_spec():
    return pl.BlockSpec((NC, BLK, D), lambda i: (0, i, 0))


def _full_spec(r, c):
    return pl.BlockSpec((r, c), lambda i: (0, 0))


_tc1 = pl.pallas_call(
    _tc1_body,
    grid=(GRID,),
    in_specs=[_dp_spec(), _row_spec(), _full_spec(D, D)],
    out_specs=_row_spec(),
    out_shape=jax.ShapeDtypeStruct((NROWS, D), jnp.float32),
    compiler_params=pltpu.CompilerParams(dimension_semantics=("arbitrary",)),
)

_tc_mid = pl.pallas_call(
    _tc_mid_body,
    grid=(GRID,),
    in_specs=[_dp_spec(), _s_spec(), _row_spec(), _full_spec(1, D),
              _full_spec(D, D)],
    out_specs=_row_spec(),
    out_shape=jax.ShapeDtypeStruct((NROWS, D), jnp.float32),
    compiler_params=pltpu.CompilerParams(dimension_semantics=("arbitrary",)),
)

_tc_fin = pl.pallas_call(
    _tc_fin_body,
    grid=(GRID,),
    in_specs=[_dp_spec(), _s_spec(), _row_spec(), _full_spec(1, D),
              pl.BlockSpec((BLK, 1), lambda i: (i, 0)),
              _full_spec(D, D), _full_spec(1, D)],
    out_specs=[_full_spec(NG, D), _full_spec(NG, D)],
    out_shape=[jax.ShapeDtypeStruct((NG, D), jnp.float32),
               jax.ShapeDtypeStruct((NG, D), jnp.float32)],
    scratch_shapes=[pltpu.VMEM((NG, 8), jnp.float32)],
    compiler_params=pltpu.CompilerParams(dimension_semantics=("arbitrary",)),
)


def kernel(x, edge_index, batch, W1, b1, W2, b2, W3, b3, Wlin, blin):
    src = edge_index[0].astype(jnp.int32)
    dst = edge_index[1].astype(jnp.int32)
    srcp = jnp.concatenate([src, jnp.zeros((EPAD - E,), jnp.int32)])
    dstp = jnp.concatenate([dst, jnp.full((EPAD - E,), N, jnp.int32)])
    src128 = srcp.reshape(EPAD // 128, 128)
    dst128 = dstp.reshape(EPAD // 128, 128)
    xp = jnp.pad(x, ((0, NROWS - N), (0, 0)))
    batp = jnp.concatenate(
        [batch.astype(jnp.int32),
         jnp.full((NROWS - N,), -1, jnp.int32)]).reshape(NROWS, 1)
    b1r = b1.reshape(1, D)
    b2r = b2.reshape(1, D)
    b3r = b3.reshape(1, D)
    wl = jnp.pad(Wlin, ((0, 0), (0, D - Wlin.shape[1])))
    blr = jnp.pad(blin, (0, D - blin.shape[0])).reshape(1, D)

    dp = _deg_kernel(jnp.ones((128, DEGW), jnp.float32), dst128)
    g1 = _tc1(dp, xp, W1)
    s1 = _scat_kernel(g1, src128, dst128)
    g2 = _tc_mid(dp, s1, g1, b1r, W2)
    s2 = _scat_kernel(g2, src128, dst128)
    g3 = _tc_mid(dp, s2, g2, b2r, W3)
    s3 = _scat_kernel(g3, src128, dst128)
    emb, logp = _tc_fin(dp, s3, g3, b3r, batp, wl, blr)
    return (logp[:, :Wlin.shape[1]], emb)
